# baseline (speedup 1.0000x reference)
import functools

import jax
import jax.numpy as jnp
from jax import lax
from jax.experimental import pallas as pl
from jax.experimental.pallas import tpu as pltpu
from jax.experimental.pallas import tpu_sc as plsc

_NUM_CORES = 2
_NUM_SUBCORES = 16
_NW = _NUM_CORES * _NUM_SUBCORES
_LANES = 16


def _dense_body(c_ref, w_ref, hp_ref, hm_ref, p_ref, q_ref, s1_ref, s2_ref):
    c_row = c_ref[...]
    u_row = lax.dot_general(c_row, w_ref[...], (((1,), (1,)), ((), ())),
                            preferred_element_type=jnp.float32)
    m_t = jnp.concatenate([c_row, u_row], axis=0)
    zp = lax.dot_general(m_t, hp_ref[...], (((1,), (1,)), ((), ())),
                         preferred_element_type=jnp.float32)
    zm = lax.dot_general(m_t, hm_ref[...], (((1,), (1,)), ((), ())),
                         preferred_element_type=jnp.float32)
    p_ref[...] = zp[0, :]
    s1_ref[...] = zp[1, :]
    q_ref[...] = zm[0, :]
    s2_ref[...] = zm[1, :]


def _sc_body(chunk, e, p_hbm, q_hbm, ei_hbm, ew_hbm, zero_hbm,
             part1_hbm, part2_hbm,
             edg_v, col_v, row_v, ew_v, pv, qv, v1, v2, p_s, q_s, acc1, acc2,
             sem_s, sem_p, sem_q):
    cid = lax.axis_index("c")
    sid = lax.axis_index("s")
    wid = cid * _NUM_SUBCORES + sid
    lo = wid * chunk
    hi = jnp.minimum(lo + chunk, e)
    start = pl.multiple_of(jnp.minimum(lo, e - chunk), 128)

    cp_e = pltpu.async_copy(ei_hbm.at[:, pl.ds(start, chunk)], edg_v, sem_s)
    cp_w = pltpu.async_copy(ew_hbm.at[pl.ds(start, chunk)], ew_v, sem_s)

    @pl.when(sid == 0)
    def _init0():
        pltpu.sync_copy(zero_hbm, acc1)
        pltpu.sync_copy(p_hbm, p_s)

    @pl.when(sid == 1)
    def _init1():
        pltpu.sync_copy(zero_hbm, acc2)
        pltpu.sync_copy(q_hbm, q_s)

    cp_e.wait()
    cp_w.wait()
    def _cpy(j, carry):
        sl = pl.ds(pl.multiple_of(j * _LANES, _LANES), _LANES)
        row_v[sl] = edg_v[0, sl]
        col_v[sl] = edg_v[1, sl]
        return carry

    lax.fori_loop(0, chunk // _LANES, _cpy, 0)
    plsc.subcore_barrier()

    cp_p = pltpu.async_copy(p_s.at[col_v], pv, sem_p)
    cp_q = pltpu.async_copy(q_s.at[col_v], qv, sem_q)
    cp_p.wait()
    cp_q.wait()

    lanes = lax.broadcasted_iota(jnp.int32, (_LANES,), 0)

    def _vec(j, carry):
        sl = pl.ds(pl.multiple_of(j * _LANES, _LANES), _LANES)
        gidx = start + j * _LANES + lanes
        wv = jnp.where((gidx >= lo) & (gidx < hi), ew_v[sl], 0.0)
        v1[sl] = wv * pv[sl]
        v2[sl] = wv * qv[sl]
        return carry

    lax.fori_loop(0, chunk // _LANES, _vec, 0)

    pltpu.sync_copy(v1, acc1.at[row_v], add=True)
    pltpu.sync_copy(v2, acc2.at[row_v], add=True)
    plsc.subcore_barrier()

    @pl.when(sid == 0)
    def _flush0():
        pltpu.sync_copy(acc1, part1_hbm.at[cid])

    @pl.when(sid == 1)
    def _flush1():
        pltpu.sync_copy(acc2, part2_hbm.at[cid])


def _combine_body(b_ref, s1_ref, s2_ref, p1_ref, p2_ref, out_ref):
    n = p1_ref.shape[1]
    bval = b_ref[0]
    out_ref[0, pl.ds(0, n)] = s1_ref[pl.ds(0, n)] + bval
    out_ref[0, pl.ds(n, n)] = p1_ref[0, :] + p1_ref[1, :]
    out_ref[0, pl.ds(2 * n, n)] = s2_ref[pl.ds(0, n)] + bval
    out_ref[0, pl.ds(3 * n, n)] = p2_ref[0, :] + p2_ref[1, :]


def kernel(c, h_pl, h_mi, edge_index, edge_weight, W, b):
    n = h_pl.shape[1]
    n_h = h_pl.shape[2]
    e = edge_weight.shape[0]

    hp = h_pl.reshape(n, n_h)
    hm = h_mi.reshape(n, n_h)
    w2 = W.reshape(n_h, n_h)

    bn = 1024
    n_blocks = -(-n // bn)
    n_pad = n_blocks * bn
    p, q, s1, s2 = pl.pallas_call(
        _dense_body,
        grid=(n_blocks,),
        in_specs=[
            pl.BlockSpec((1, n_h), lambda i: (0, 0)),
            pl.BlockSpec((n_h, n_h), lambda i: (0, 0)),
            pl.BlockSpec((bn, n_h), lambda i: (i, 0)),
            pl.BlockSpec((bn, n_h), lambda i: (i, 0)),
        ],
        out_specs=[
            pl.BlockSpec((bn,), lambda i: (i,)),
            pl.BlockSpec((bn,), lambda i: (i,)),
            pl.BlockSpec((bn,), lambda i: (i,)),
            pl.BlockSpec((bn,), lambda i: (i,)),
        ],
        out_shape=[
            jax.ShapeDtypeStruct((n_pad,), jnp.float32),
            jax.ShapeDtypeStruct((n_pad,), jnp.float32),
            jax.ShapeDtypeStruct((n_pad,), jnp.float32),
            jax.ShapeDtypeStruct((n_pad,), jnp.float32),
        ],
    )(c, w2, hp, hm)

    if e % 128 == 0:
        ei = edge_index
        ew = edge_weight
        e_pad = e
    else:
        e_pad = -(-e // 128) * 128
        pad = e_pad - e
        ei = jnp.concatenate(
            [edge_index, jnp.zeros((2, pad), edge_index.dtype)], axis=1)
        ew = jnp.concatenate([edge_weight, jnp.zeros((pad,), jnp.float32)])
    chunk = -(-e_pad // (_NW * 128)) * 128
    zero = jnp.zeros((n,), jnp.float32)

    sc = pl.kernel(
        functools.partial(_sc_body, chunk, e_pad),
        out_type=[jax.ShapeDtypeStruct((_NUM_CORES, n), jnp.float32),
                  jax.ShapeDtypeStruct((_NUM_CORES, n), jnp.float32)],
        mesh=plsc.VectorSubcoreMesh(core_axis_name="c", subcore_axis_name="s"),
        scratch_types=[
            pltpu.VMEM((2, chunk), jnp.int32),
            pltpu.VMEM((chunk,), jnp.int32),
            pltpu.VMEM((chunk,), jnp.int32),
            pltpu.VMEM((chunk,), jnp.float32),
            pltpu.VMEM((chunk,), jnp.float32),
            pltpu.VMEM((chunk,), jnp.float32),
            pltpu.VMEM((chunk,), jnp.float32),
            pltpu.VMEM((chunk,), jnp.float32),
            pltpu.VMEM_SHARED((n_pad,), jnp.float32),
            pltpu.VMEM_SHARED((n_pad,), jnp.float32),
            pltpu.VMEM_SHARED((n,), jnp.float32),
            pltpu.VMEM_SHARED((n,), jnp.float32),
            pltpu.SemaphoreType.DMA,
            pltpu.SemaphoreType.DMA,
            pltpu.SemaphoreType.DMA,
        ],
    )
    part1, part2 = sc(p, q, ei, ew, zero)

    logits = pl.pallas_call(
        _combine_body,
        in_specs=[
            pl.BlockSpec(memory_space=pltpu.SMEM),
            pl.BlockSpec(memory_space=pltpu.VMEM),
            pl.BlockSpec(memory_space=pltpu.VMEM),
            pl.BlockSpec(memory_space=pltpu.VMEM),
            pl.BlockSpec(memory_space=pltpu.VMEM),
        ],
        out_specs=pl.BlockSpec(memory_space=pltpu.VMEM),
        out_shape=jax.ShapeDtypeStruct((1, 4 * n), jnp.float32),
    )(b, s1, s2, part1, part2)
    return logits


# speedup vs baseline: 1.0768x; 1.0768x over previous
import functools

import jax
import jax.numpy as jnp
from jax import lax
from jax.experimental import pallas as pl
from jax.experimental.pallas import tpu as pltpu
from jax.experimental.pallas import tpu_sc as plsc

_NUM_CORES = 2
_NUM_SUBCORES = 16
_NW = _NUM_CORES * _NUM_SUBCORES
_LANES = 16


def _dense_body(c_ref, w_ref, hp_ref, hm_ref, p_ref, q_ref, s1_ref, s2_ref):
    c_row = c_ref[...]
    u_row = lax.dot_general(c_row, w_ref[...], (((1,), (1,)), ((), ())),
                            preferred_element_type=jnp.float32)
    m_t = jnp.concatenate([c_row, u_row], axis=0)
    zp = lax.dot_general(m_t, hp_ref[...], (((1,), (1,)), ((), ())),
                         preferred_element_type=jnp.float32)
    zm = lax.dot_general(m_t, hm_ref[...], (((1,), (1,)), ((), ())),
                         preferred_element_type=jnp.float32)
    p_ref[...] = zp[0, :]
    s1_ref[...] = zp[1, :]
    q_ref[...] = zm[0, :]
    s2_ref[...] = zm[1, :]


def _sc_body(chunk, e, p_hbm, q_hbm, ei_hbm, ew_hbm, zero_hbm,
             part1_hbm, part2_hbm,
             edg_v, col_v, row_v, ew_v, pv, qv, v1, v2, p_s, q_s, acc1, acc2,
             sem_s, sem_p, sem_q):
    cid = lax.axis_index("c")
    sid = lax.axis_index("s")
    wid = cid * _NUM_SUBCORES + sid
    lo = wid * chunk
    hi = jnp.minimum(lo + chunk, e)
    start = pl.multiple_of(jnp.minimum(lo, e - chunk), 128)

    cp_e = pltpu.async_copy(ei_hbm.at[:, pl.ds(start, chunk)], edg_v, sem_s)
    cp_w = pltpu.async_copy(ew_hbm.at[pl.ds(start, chunk)], ew_v, sem_s)

    @pl.when(sid == 0)
    def _init0():
        pltpu.sync_copy(zero_hbm, acc1)
        pltpu.sync_copy(p_hbm, p_s)

    @pl.when(sid == 1)
    def _init1():
        pltpu.sync_copy(zero_hbm, acc2)
        pltpu.sync_copy(q_hbm, q_s)

    cp_e.wait()
    cp_w.wait()
    def _cpy(j, carry):
        sl = pl.ds(pl.multiple_of(j * _LANES, _LANES), _LANES)
        row_v[sl] = edg_v[0, sl]
        col_v[sl] = edg_v[1, sl]
        return carry

    lax.fori_loop(0, chunk // _LANES, _cpy, 0)
    plsc.subcore_barrier()

    cp_p = pltpu.async_copy(p_s.at[col_v], pv, sem_p)
    cp_q = pltpu.async_copy(q_s.at[col_v], qv, sem_q)
    cp_p.wait()
    cp_q.wait()

    lanes = lax.broadcasted_iota(jnp.int32, (_LANES,), 0)

    def _vec(j, carry):
        sl = pl.ds(pl.multiple_of(j * _LANES, _LANES), _LANES)
        gidx = start + j * _LANES + lanes
        wv = jnp.where((gidx >= lo) & (gidx < hi), ew_v[sl], 0.0)
        v1[sl] = wv * pv[sl]
        v2[sl] = wv * qv[sl]
        return carry

    lax.fori_loop(0, chunk // _LANES, _vec, 0)

    pltpu.sync_copy(v1, acc1.at[row_v], add=True)
    pltpu.sync_copy(v2, acc2.at[row_v], add=True)
    plsc.subcore_barrier()

    @pl.when(sid == 0)
    def _flush0():
        pltpu.sync_copy(acc1, part1_hbm.at[cid])

    @pl.when(sid == 1)
    def _flush1():
        pltpu.sync_copy(acc2, part2_hbm.at[cid])


def _combine_body(b_ref, s1_ref, s2_ref, p1_ref, p2_ref, out_ref):
    n = p1_ref.shape[1]
    bval = b_ref[0]
    out_ref[0, pl.ds(0, n)] = s1_ref[pl.ds(0, n)] + bval
    out_ref[0, pl.ds(n, n)] = p1_ref[0, :] + p1_ref[1, :]
    out_ref[0, pl.ds(2 * n, n)] = s2_ref[pl.ds(0, n)] + bval
    out_ref[0, pl.ds(3 * n, n)] = p2_ref[0, :] + p2_ref[1, :]


def kernel(c, h_pl, h_mi, edge_index, edge_weight, W, b):
    n = h_pl.shape[1]
    n_h = h_pl.shape[2]
    e = edge_weight.shape[0]

    hp = h_pl.reshape(n, n_h)
    hm = h_mi.reshape(n, n_h)
    w2 = W.reshape(n_h, n_h)

    bn = 5120
    n_blocks = -(-n // bn)
    n_pad = n_blocks * bn
    p, q, s1, s2 = pl.pallas_call(
        _dense_body,
        grid=(n_blocks,),
        in_specs=[
            pl.BlockSpec((1, n_h), lambda i: (0, 0)),
            pl.BlockSpec((n_h, n_h), lambda i: (0, 0)),
            pl.BlockSpec((bn, n_h), lambda i: (i, 0)),
            pl.BlockSpec((bn, n_h), lambda i: (i, 0)),
        ],
        out_specs=[
            pl.BlockSpec((bn,), lambda i: (i,)),
            pl.BlockSpec((bn,), lambda i: (i,)),
            pl.BlockSpec((bn,), lambda i: (i,)),
            pl.BlockSpec((bn,), lambda i: (i,)),
        ],
        out_shape=[
            jax.ShapeDtypeStruct((n_pad,), jnp.float32),
            jax.ShapeDtypeStruct((n_pad,), jnp.float32),
            jax.ShapeDtypeStruct((n_pad,), jnp.float32),
            jax.ShapeDtypeStruct((n_pad,), jnp.float32),
        ],
    )(c, w2, hp, hm)

    if e % 128 == 0:
        ei = edge_index
        ew = edge_weight
        e_pad = e
    else:
        e_pad = -(-e // 128) * 128
        pad = e_pad - e
        ei = jnp.concatenate(
            [edge_index, jnp.zeros((2, pad), edge_index.dtype)], axis=1)
        ew = jnp.concatenate([edge_weight, jnp.zeros((pad,), jnp.float32)])
    chunk = -(-e_pad // (_NW * 128)) * 128
    zero = jnp.zeros((n,), jnp.float32)

    sc = pl.kernel(
        functools.partial(_sc_body, chunk, e_pad),
        out_type=[jax.ShapeDtypeStruct((_NUM_CORES, n), jnp.float32),
                  jax.ShapeDtypeStruct((_NUM_CORES, n), jnp.float32)],
        mesh=plsc.VectorSubcoreMesh(core_axis_name="c", subcore_axis_name="s"),
        scratch_types=[
            pltpu.VMEM((2, chunk), jnp.int32),
            pltpu.VMEM((chunk,), jnp.int32),
            pltpu.VMEM((chunk,), jnp.int32),
            pltpu.VMEM((chunk,), jnp.float32),
            pltpu.VMEM((chunk,), jnp.float32),
            pltpu.VMEM((chunk,), jnp.float32),
            pltpu.VMEM((chunk,), jnp.float32),
            pltpu.VMEM((chunk,), jnp.float32),
            pltpu.VMEM_SHARED((n_pad,), jnp.float32),
            pltpu.VMEM_SHARED((n_pad,), jnp.float32),
            pltpu.VMEM_SHARED((n,), jnp.float32),
            pltpu.VMEM_SHARED((n,), jnp.float32),
            pltpu.SemaphoreType.DMA,
            pltpu.SemaphoreType.DMA,
            pltpu.SemaphoreType.DMA,
        ],
    )
    part1, part2 = sc(p, q, ei, ew, zero)

    logits = pl.pallas_call(
        _combine_body,
        in_specs=[
            pl.BlockSpec(memory_space=pltpu.SMEM),
            pl.BlockSpec(memory_space=pltpu.VMEM),
            pl.BlockSpec(memory_space=pltpu.VMEM),
            pl.BlockSpec(memory_space=pltpu.VMEM),
            pl.BlockSpec(memory_space=pltpu.VMEM),
        ],
        out_specs=pl.BlockSpec(memory_space=pltpu.VMEM),
        out_shape=jax.ShapeDtypeStruct((1, 4 * n), jnp.float32),
    )(b, s1, s2, part1, part2)
    return logits
